# baseline (speedup 1.0000x reference)
import jax
import jax.numpy as jnp
from jax import lax
from jax.experimental import pallas as pl
from jax.experimental.pallas import tpu as pltpu
from jax.experimental.pallas import tpu_sc as plsc

_NT = 16
_K = 128
_IC = 32
_HALF = 128


def _make_scatter(NP, EP):
    chunk = EP // _NT
    n_it = chunk // _K
    nq = n_it // _IC
    n2 = _IC // 2
    rpt = NP // _NT

    mesh = plsc.VectorSubcoreMesh(core_axis_name="c", subcore_axis_name="s")

    def body(g3, src_h, dst_h, acc3, isrc, idst, rows_a, rows_b,
             acc_sh, sem_a, sem_b, sem_sa, sem_sb):
        c = lax.axis_index("c")
        s = lax.axis_index("s")
        r0 = s * rpt
        row0 = s * n_it

        pltpu.sync_copy(g3.at[c, pl.ds(r0, rpt)], acc_sh.at[pl.ds(r0, rpt)])
        plsc.subcore_barrier()
        gsrc = g3.at[c]

        def qloop(q, carry):
            pltpu.sync_copy(src_h.at[pl.ds(row0 + q * _IC, _IC)], isrc)
            pltpu.sync_copy(dst_h.at[pl.ds(row0 + q * _IC, _IC)], idst)
            pltpu.async_copy(gsrc.at[isrc.at[0]], rows_a, sem_a)
            pltpu.async_copy(gsrc.at[isrc.at[1]], rows_b, sem_b)

            def it(j, carry2):
                i0 = 2 * j
                pltpu.make_async_copy(gsrc.at[isrc.at[i0]],
                                      rows_a, sem_a).wait()
                pltpu.async_copy(rows_a, acc_sh.at[idst.at[i0]],
                                 sem_sa, add=True)
                pltpu.make_async_copy(gsrc.at[isrc.at[i0 + 1]],
                                      rows_b, sem_b).wait()
                pltpu.async_copy(rows_b, acc_sh.at[idst.at[i0 + 1]],
                                 sem_sb, add=True)
                pltpu.make_async_copy(rows_a, acc_sh.at[idst.at[i0]],
                                      sem_sa).wait()
                pltpu.make_async_copy(rows_b, acc_sh.at[idst.at[i0 + 1]],
                                      sem_sb).wait()

                @pl.when(j + 1 < n2)
                def _():
                    pltpu.async_copy(gsrc.at[isrc.at[i0 + 2]], rows_a, sem_a)
                    pltpu.async_copy(gsrc.at[isrc.at[i0 + 3]], rows_b, sem_b)

                return carry2

            lax.fori_loop(0, n2, it, 0)
            return carry

        lax.fori_loop(0, nq, qloop, 0)
        plsc.subcore_barrier()
        pltpu.sync_copy(acc_sh.at[pl.ds(r0, rpt)], acc3.at[c, pl.ds(r0, rpt)])

    return pl.kernel(
        body,
        out_type=jax.ShapeDtypeStruct((2, NP, _HALF), jnp.float32),
        mesh=mesh,
        scratch_types=[
            pltpu.VMEM((_IC, _K), jnp.int32),
            pltpu.VMEM((_IC, _K), jnp.int32),
            pltpu.VMEM((_K, _HALF), jnp.float32),
            pltpu.VMEM((_K, _HALF), jnp.float32),
            pltpu.VMEM_SHARED((NP, _HALF), jnp.float32),
            pltpu.SemaphoreType.DMA,
            pltpu.SemaphoreType.DMA,
            pltpu.SemaphoreType.DMA,
            pltpu.SemaphoreType.DMA,
        ],
    )


def _make_deg(NP, EP):
    half = EP // 2
    chunk = half // _NT
    n_it = chunk // _K
    rpt = NP // _NT

    mesh = plsc.VectorSubcoreMesh(core_axis_name="c", subcore_axis_name="s")

    def body(dst_h, ones_h, zeros_h, deg3, ones_v, idx_d, deg_sh, sem):
        c = lax.axis_index("c")
        s = lax.axis_index("s")
        r0 = s * rpt
        pltpu.sync_copy(zeros_h.at[pl.ds(r0, rpt)], deg_sh.at[pl.ds(r0, rpt)])
        pltpu.sync_copy(ones_h, ones_v)
        plsc.subcore_barrier()
        e0 = c * half + s * chunk

        def it(i, carry):
            base = e0 + i * _K
            pltpu.sync_copy(dst_h.at[pl.ds(base, _K)], idx_d)
            pltpu.sync_copy(ones_v, deg_sh.at[idx_d], add=True)
            return carry

        lax.fori_loop(0, n_it, it, 0)
        plsc.subcore_barrier()
        pltpu.sync_copy(deg_sh.at[pl.ds(r0, rpt)], deg3.at[c, pl.ds(r0, rpt)])

    return pl.kernel(
        body,
        out_type=jax.ShapeDtypeStruct((2, NP, _HALF), jnp.float32),
        mesh=mesh,
        scratch_types=[
            pltpu.VMEM((_K, _HALF), jnp.float32),
            pltpu.VMEM((_K,), jnp.int32),
            pltpu.VMEM_SHARED((NP, _HALF), jnp.float32),
            pltpu.SemaphoreType.DMA,
        ],
    )


def _dis(deg3_ref):
    return lax.rsqrt(deg3_ref[0, :, 0:1] + deg3_ref[1, :, 0:1] + 1.0)


def _enc_body(x_ref, we_ref, be_ref, w0_ref, deg3_ref, g3_ref):
    dis = _dis(deg3_ref)
    h = jnp.dot(x_ref[...], we_ref[...],
                preferred_element_type=jnp.float32) + be_ref[...]
    g = dis * jnp.dot(h, w0_ref[...], preferred_element_type=jnp.float32)
    g3_ref[0] = g[:, :_HALF]
    g3_ref[1] = g[:, _HALF:]


def _mid_body(acc3_ref, deg3_ref, b_ref, w_ref, g3_ref):
    dis = _dis(deg3_ref)
    h_a = jnp.maximum(dis * acc3_ref[0] + b_ref[:, :_HALF], 0.0)
    h_b = jnp.maximum(dis * acc3_ref[1] + b_ref[:, _HALF:], 0.0)
    h = jnp.concatenate([h_a, h_b], axis=1)
    g = dis * jnp.dot(h, w_ref[...], preferred_element_type=jnp.float32)
    g3_ref[0] = g[:, :_HALF]
    g3_ref[1] = g[:, _HALF:]


def _make_fin(N):
    def body(acc3_ref, deg3_ref, b_ref, out_ref):
        dis = _dis(deg3_ref)
        h_a = jnp.maximum(dis * acc3_ref[0] + b_ref[:, :_HALF], 0.0)
        h_b = jnp.maximum(dis * acc3_ref[1] + b_ref[:, _HALF:], 0.0)
        out_ref[...] = jnp.concatenate([h_a, h_b], axis=1)[:N]

    return pl.pallas_call(
        body,
        out_shape=jax.ShapeDtypeStruct((N, 2 * _HALF), jnp.float32),
    )


@jax.jit
def kernel(x, edge_index, W_enc, b_enc, W0, b0, W1, b1, W2, b2, W3, b3):
    N, D_IN = x.shape
    E = edge_index.shape[1]

    NP = ((N + 1 + 127) // 128) * 128
    grain = _NT * _K * _IC
    EP = ((E + grain - 1) // grain) * grain

    x_p = jnp.pad(x, ((0, NP - N), (0, 0)))
    pad = jnp.full((EP - E,), N, jnp.int32)
    src_p = jnp.concatenate([edge_index[0], pad])
    dst_p = jnp.concatenate([edge_index[1], pad])

    ones_h = jnp.ones((_K, _HALF), jnp.float32)
    zeros_h = jnp.zeros((NP, _HALF), jnp.float32)

    deg3 = _make_deg(NP, EP)(dst_p, ones_h, zeros_h)

    enc = pl.pallas_call(
        _enc_body,
        out_shape=jax.ShapeDtypeStruct((2, NP, _HALF), jnp.float32),
    )
    mid = pl.pallas_call(
        _mid_body,
        out_shape=jax.ShapeDtypeStruct((2, NP, _HALF), jnp.float32),
    )
    scatter = _make_scatter(NP, EP)

    src2 = src_p.reshape(-1, _K)
    dst2 = dst_p.reshape(-1, _K)

    g3 = enc(x_p, W_enc, b_enc.reshape(1, -1), W0, deg3)
    acc3 = scatter(g3, src2, dst2)
    for b_prev, W_next in ((b0, W1), (b1, W2), (b2, W3)):
        g3 = mid(acc3, deg3, b_prev.reshape(1, -1), W_next)
        acc3 = scatter(g3, src2, dst2)
    return _make_fin(N)(acc3, deg3, b3.reshape(1, -1))


# speedup vs baseline: 1.0508x; 1.0508x over previous
import jax
import jax.numpy as jnp
from jax import lax
from jax.experimental import pallas as pl
from jax.experimental.pallas import tpu as pltpu
from jax.experimental.pallas import tpu_sc as plsc

_NT = 16
_K = 128
_IC = 16
_HALF = 128


def _make_scatter(NP, EP):
    chunk = EP // _NT
    n_it = chunk // _K
    nq = n_it // _IC
    n2 = _IC // 2
    rpt = NP // _NT

    mesh = plsc.VectorSubcoreMesh(core_axis_name="c", subcore_axis_name="s")

    def body(g3, src_h, dst_h, acc3, isrc2, idst2, rows_a, rows_b,
             acc_sh, sem_i, sem_a, sem_b, sem_sa, sem_sb):
        c = lax.axis_index("c")
        s = lax.axis_index("s")
        r0 = s * rpt
        row0 = s * n_it

        pltpu.async_copy(src_h.at[pl.ds(row0, _IC)], isrc2.at[0], sem_i)
        pltpu.async_copy(dst_h.at[pl.ds(row0, _IC)], idst2.at[0], sem_i)
        pltpu.sync_copy(g3.at[c, pl.ds(r0, rpt)], acc_sh.at[pl.ds(r0, rpt)])
        plsc.subcore_barrier()
        gsrc = g3.at[c]

        def qloop(q, carry):
            p = lax.rem(q, 2)
            isrc = isrc2.at[p]
            idst = idst2.at[p]
            pltpu.make_async_copy(src_h.at[pl.ds(row0, _IC)],
                                  isrc, sem_i).wait()
            pltpu.make_async_copy(dst_h.at[pl.ds(row0, _IC)],
                                  idst, sem_i).wait()

            @pl.when(q + 1 < nq)
            def _():
                nxt = row0 + (q + 1) * _IC
                pltpu.async_copy(src_h.at[pl.ds(nxt, _IC)],
                                 isrc2.at[1 - p], sem_i)
                pltpu.async_copy(dst_h.at[pl.ds(nxt, _IC)],
                                 idst2.at[1 - p], sem_i)

            pltpu.async_copy(gsrc.at[isrc.at[0]], rows_a, sem_a)
            pltpu.async_copy(gsrc.at[isrc.at[1]], rows_b, sem_b)

            def it(j, carry2):
                i0 = 2 * j
                pltpu.make_async_copy(gsrc.at[isrc.at[i0]],
                                      rows_a, sem_a).wait()
                pltpu.async_copy(rows_a, acc_sh.at[idst.at[i0]],
                                 sem_sa, add=True)
                pltpu.make_async_copy(gsrc.at[isrc.at[i0 + 1]],
                                      rows_b, sem_b).wait()
                pltpu.async_copy(rows_b, acc_sh.at[idst.at[i0 + 1]],
                                 sem_sb, add=True)
                pltpu.make_async_copy(rows_a, acc_sh.at[idst.at[i0]],
                                      sem_sa).wait()

                @pl.when(j + 1 < n2)
                def _():
                    pltpu.async_copy(gsrc.at[isrc.at[i0 + 2]], rows_a, sem_a)

                pltpu.make_async_copy(rows_b, acc_sh.at[idst.at[i0 + 1]],
                                      sem_sb).wait()

                @pl.when(j + 1 < n2)
                def _():
                    pltpu.async_copy(gsrc.at[isrc.at[i0 + 3]], rows_b, sem_b)

                return carry2

            lax.fori_loop(0, n2, it, 0)
            return carry

        lax.fori_loop(0, nq, qloop, 0)
        plsc.subcore_barrier()
        pltpu.sync_copy(acc_sh.at[pl.ds(r0, rpt)], acc3.at[c, pl.ds(r0, rpt)])

    return pl.kernel(
        body,
        out_type=jax.ShapeDtypeStruct((2, NP, _HALF), jnp.float32),
        mesh=mesh,
        scratch_types=[
            pltpu.VMEM((2, _IC, _K), jnp.int32),
            pltpu.VMEM((2, _IC, _K), jnp.int32),
            pltpu.VMEM((_K, _HALF), jnp.float32),
            pltpu.VMEM((_K, _HALF), jnp.float32),
            pltpu.VMEM_SHARED((NP, _HALF), jnp.float32),
            pltpu.SemaphoreType.DMA,
            pltpu.SemaphoreType.DMA,
            pltpu.SemaphoreType.DMA,
            pltpu.SemaphoreType.DMA,
            pltpu.SemaphoreType.DMA,
        ],
    )


def _make_deg(NP, EP):
    half = EP // 2
    chunk = half // _NT
    n_it = chunk // _K
    rpt = NP // _NT

    mesh = plsc.VectorSubcoreMesh(core_axis_name="c", subcore_axis_name="s")

    def body(dst_h, ones_h, zeros_h, deg3, ones_v, idx_d, deg_sh, sem):
        c = lax.axis_index("c")
        s = lax.axis_index("s")
        r0 = s * rpt
        pltpu.sync_copy(zeros_h.at[pl.ds(r0, rpt)], deg_sh.at[pl.ds(r0, rpt)])
        pltpu.sync_copy(ones_h, ones_v)
        plsc.subcore_barrier()
        e0 = c * half + s * chunk

        def it(i, carry):
            base = e0 + i * _K
            pltpu.sync_copy(dst_h.at[pl.ds(base, _K)], idx_d)
            pltpu.sync_copy(ones_v, deg_sh.at[idx_d], add=True)
            return carry

        lax.fori_loop(0, n_it, it, 0)
        plsc.subcore_barrier()
        pltpu.sync_copy(deg_sh.at[pl.ds(r0, rpt)], deg3.at[c, pl.ds(r0, rpt)])

    return pl.kernel(
        body,
        out_type=jax.ShapeDtypeStruct((2, NP, _HALF), jnp.float32),
        mesh=mesh,
        scratch_types=[
            pltpu.VMEM((_K, _HALF), jnp.float32),
            pltpu.VMEM((_K,), jnp.int32),
            pltpu.VMEM_SHARED((NP, _HALF), jnp.float32),
            pltpu.SemaphoreType.DMA,
        ],
    )


def _dis(deg3_ref):
    return lax.rsqrt(deg3_ref[0, :, 0:1] + deg3_ref[1, :, 0:1] + 1.0)


def _enc_body(x_ref, we_ref, be_ref, w0_ref, deg3_ref, g3_ref):
    dis = _dis(deg3_ref)
    h = jnp.dot(x_ref[...], we_ref[...],
                preferred_element_type=jnp.float32) + be_ref[...]
    g = dis * jnp.dot(h, w0_ref[...], preferred_element_type=jnp.float32)
    g3_ref[0] = g[:, :_HALF]
    g3_ref[1] = g[:, _HALF:]


def _mid_body(acc3_ref, deg3_ref, b_ref, w_ref, g3_ref):
    dis = _dis(deg3_ref)
    h_a = jnp.maximum(dis * acc3_ref[0] + b_ref[:, :_HALF], 0.0)
    h_b = jnp.maximum(dis * acc3_ref[1] + b_ref[:, _HALF:], 0.0)
    h = jnp.concatenate([h_a, h_b], axis=1)
    g = dis * jnp.dot(h, w_ref[...], preferred_element_type=jnp.float32)
    g3_ref[0] = g[:, :_HALF]
    g3_ref[1] = g[:, _HALF:]


def _make_fin(N):
    def body(acc3_ref, deg3_ref, b_ref, out_ref):
        dis = _dis(deg3_ref)
        h_a = jnp.maximum(dis * acc3_ref[0] + b_ref[:, :_HALF], 0.0)
        h_b = jnp.maximum(dis * acc3_ref[1] + b_ref[:, _HALF:], 0.0)
        out_ref[...] = jnp.concatenate([h_a, h_b], axis=1)[:N]

    return pl.pallas_call(
        body,
        out_shape=jax.ShapeDtypeStruct((N, 2 * _HALF), jnp.float32),
    )


@jax.jit
def kernel(x, edge_index, W_enc, b_enc, W0, b0, W1, b1, W2, b2, W3, b3):
    N, D_IN = x.shape
    E = edge_index.shape[1]

    NP = ((N + 1 + 127) // 128) * 128
    grain = _NT * _K * _IC
    EP = ((E + grain - 1) // grain) * grain

    x_p = jnp.pad(x, ((0, NP - N), (0, 0)))
    pad = jnp.full((EP - E,), N, jnp.int32)
    src_p = jnp.concatenate([edge_index[0], pad])
    dst_p = jnp.concatenate([edge_index[1], pad])

    ones_h = jnp.ones((_K, _HALF), jnp.float32)
    zeros_h = jnp.zeros((NP, _HALF), jnp.float32)

    deg3 = _make_deg(NP, EP)(dst_p, ones_h, zeros_h)

    enc = pl.pallas_call(
        _enc_body,
        out_shape=jax.ShapeDtypeStruct((2, NP, _HALF), jnp.float32),
    )
    mid = pl.pallas_call(
        _mid_body,
        out_shape=jax.ShapeDtypeStruct((2, NP, _HALF), jnp.float32),
    )
    scatter = _make_scatter(NP, EP)

    src2 = src_p.reshape(-1, _K)
    dst2 = dst_p.reshape(-1, _K)

    g3 = enc(x_p, W_enc, b_enc.reshape(1, -1), W0, deg3)
    acc3 = scatter(g3, src2, dst2)
    for b_prev, W_next in ((b0, W1), (b1, W2), (b2, W3)):
        g3 = mid(acc3, deg3, b_prev.reshape(1, -1), W_next)
        acc3 = scatter(g3, src2, dst2)
    return _make_fin(N)(acc3, deg3, b3.reshape(1, -1))


# speedup vs baseline: 1.1296x; 1.0751x over previous
import jax
import jax.numpy as jnp
from jax import lax
from jax.experimental import pallas as pl
from jax.experimental.pallas import tpu as pltpu
from jax.experimental.pallas import tpu_sc as plsc

_NT = 16
_K = 128
_IC = 16
_HALF = 128


def _make_scatter(NP, EP):
    chunk = EP // _NT
    n_it = chunk // _K
    nq = n_it // _IC
    n2 = _IC // 2
    rpt = NP // _NT

    mesh = plsc.VectorSubcoreMesh(core_axis_name="c", subcore_axis_name="s")

    def body(g3, src_h, dst_h, acc3, isrc2, idst2, rows_a, rows_b,
             acc_sh, sem_i, sem_a, sem_b):
        c = lax.axis_index("c")
        s = lax.axis_index("s")
        r0 = s * rpt
        row0 = s * n_it

        pltpu.async_copy(src_h.at[pl.ds(row0, _IC)], isrc2.at[0], sem_i)
        pltpu.async_copy(dst_h.at[pl.ds(row0, _IC)], idst2.at[0], sem_i)
        pltpu.sync_copy(g3.at[c, pl.ds(r0, rpt)], acc_sh.at[pl.ds(r0, rpt)])
        plsc.subcore_barrier()
        gsrc = g3.at[c]

        def qloop(q, carry):
            p = lax.rem(q, 2)
            isrc = isrc2.at[p]
            idst = idst2.at[p]
            pltpu.make_async_copy(src_h.at[pl.ds(row0, _IC)],
                                  isrc, sem_i).wait()
            pltpu.make_async_copy(dst_h.at[pl.ds(row0, _IC)],
                                  idst, sem_i).wait()

            @pl.when(q + 1 < nq)
            def _():
                nxt = row0 + (q + 1) * _IC
                pltpu.async_copy(src_h.at[pl.ds(nxt, _IC)],
                                 isrc2.at[1 - p], sem_i)
                pltpu.async_copy(dst_h.at[pl.ds(nxt, _IC)],
                                 idst2.at[1 - p], sem_i)

            pltpu.async_copy(gsrc.at[isrc.at[0]], rows_a, sem_a)
            pltpu.async_copy(gsrc.at[isrc.at[1]], rows_b, sem_b)

            def it(j, carry2):
                i0 = 2 * j
                pltpu.make_async_copy(gsrc.at[isrc.at[i0]],
                                      rows_a, sem_a).wait()
                pltpu.sync_copy(rows_a, acc_sh.at[idst.at[i0]], add=True)

                @pl.when(j + 1 < n2)
                def _():
                    pltpu.async_copy(gsrc.at[isrc.at[i0 + 2]], rows_a, sem_a)

                pltpu.make_async_copy(gsrc.at[isrc.at[i0 + 1]],
                                      rows_b, sem_b).wait()
                pltpu.sync_copy(rows_b, acc_sh.at[idst.at[i0 + 1]], add=True)

                @pl.when(j + 1 < n2)
                def _():
                    pltpu.async_copy(gsrc.at[isrc.at[i0 + 3]], rows_b, sem_b)

                return carry2

            lax.fori_loop(0, n2, it, 0)
            return carry

        lax.fori_loop(0, nq, qloop, 0)
        plsc.subcore_barrier()
        pltpu.sync_copy(acc_sh.at[pl.ds(r0, rpt)], acc3.at[c, pl.ds(r0, rpt)])

    return pl.kernel(
        body,
        out_type=jax.ShapeDtypeStruct((2, NP, _HALF), jnp.float32),
        mesh=mesh,
        scratch_types=[
            pltpu.VMEM((2, _IC, _K), jnp.int32),
            pltpu.VMEM((2, _IC, _K), jnp.int32),
            pltpu.VMEM((_K, _HALF), jnp.float32),
            pltpu.VMEM((_K, _HALF), jnp.float32),
            pltpu.VMEM_SHARED((NP, _HALF), jnp.float32),
            pltpu.SemaphoreType.DMA,
            pltpu.SemaphoreType.DMA,
            pltpu.SemaphoreType.DMA,
        ],
    )


def _make_deg(NP, EP):
    half = EP // 2
    chunk = half // _NT
    n_it = chunk // _K
    rpt = NP // _NT

    mesh = plsc.VectorSubcoreMesh(core_axis_name="c", subcore_axis_name="s")

    def body(dst_h, ones_h, zeros_h, deg3, ones_v, idx_d, deg_sh, sem):
        c = lax.axis_index("c")
        s = lax.axis_index("s")
        r0 = s * rpt
        pltpu.sync_copy(zeros_h.at[pl.ds(r0, rpt)], deg_sh.at[pl.ds(r0, rpt)])
        pltpu.sync_copy(ones_h, ones_v)
        plsc.subcore_barrier()
        e0 = c * half + s * chunk

        def it(i, carry):
            base = e0 + i * _K
            pltpu.sync_copy(dst_h.at[pl.ds(base, _K)], idx_d)
            pltpu.sync_copy(ones_v, deg_sh.at[idx_d], add=True)
            return carry

        lax.fori_loop(0, n_it, it, 0)
        plsc.subcore_barrier()
        pltpu.sync_copy(deg_sh.at[pl.ds(r0, rpt)], deg3.at[c, pl.ds(r0, rpt)])

    return pl.kernel(
        body,
        out_type=jax.ShapeDtypeStruct((2, NP, _HALF), jnp.float32),
        mesh=mesh,
        scratch_types=[
            pltpu.VMEM((_K, _HALF), jnp.float32),
            pltpu.VMEM((_K,), jnp.int32),
            pltpu.VMEM_SHARED((NP, _HALF), jnp.float32),
            pltpu.SemaphoreType.DMA,
        ],
    )


def _dis(deg3_ref):
    return lax.rsqrt(deg3_ref[0, :, 0:1] + deg3_ref[1, :, 0:1] + 1.0)


def _enc_body(x_ref, we_ref, be_ref, w0_ref, deg3_ref, g3_ref):
    dis = _dis(deg3_ref)
    h = jnp.dot(x_ref[...], we_ref[...],
                preferred_element_type=jnp.float32) + be_ref[...]
    g = dis * jnp.dot(h, w0_ref[...], preferred_element_type=jnp.float32)
    g3_ref[0] = g[:, :_HALF]
    g3_ref[1] = g[:, _HALF:]


def _mid_body(acc3_ref, deg3_ref, b_ref, w_ref, g3_ref):
    dis = _dis(deg3_ref)
    h_a = jnp.maximum(dis * acc3_ref[0] + b_ref[:, :_HALF], 0.0)
    h_b = jnp.maximum(dis * acc3_ref[1] + b_ref[:, _HALF:], 0.0)
    h = jnp.concatenate([h_a, h_b], axis=1)
    g = dis * jnp.dot(h, w_ref[...], preferred_element_type=jnp.float32)
    g3_ref[0] = g[:, :_HALF]
    g3_ref[1] = g[:, _HALF:]


def _make_fin(N):
    def body(acc3_ref, deg3_ref, b_ref, out_ref):
        dis = _dis(deg3_ref)
        h_a = jnp.maximum(dis * acc3_ref[0] + b_ref[:, :_HALF], 0.0)
        h_b = jnp.maximum(dis * acc3_ref[1] + b_ref[:, _HALF:], 0.0)
        out_ref[...] = jnp.concatenate([h_a, h_b], axis=1)[:N]

    return pl.pallas_call(
        body,
        out_shape=jax.ShapeDtypeStruct((N, 2 * _HALF), jnp.float32),
    )


@jax.jit
def kernel(x, edge_index, W_enc, b_enc, W0, b0, W1, b1, W2, b2, W3, b3):
    N, D_IN = x.shape
    E = edge_index.shape[1]

    NP = ((N + 1 + 127) // 128) * 128
    grain = _NT * _K * _IC
    EP = ((E + grain - 1) // grain) * grain

    x_p = jnp.pad(x, ((0, NP - N), (0, 0)))
    pad = jnp.full((EP - E,), N, jnp.int32)
    src_p = jnp.concatenate([edge_index[0], pad])
    dst_p = jnp.concatenate([edge_index[1], pad])

    ones_h = jnp.ones((_K, _HALF), jnp.float32)
    zeros_h = jnp.zeros((NP, _HALF), jnp.float32)

    deg3 = _make_deg(NP, EP)(dst_p, ones_h, zeros_h)

    enc = pl.pallas_call(
        _enc_body,
        out_shape=jax.ShapeDtypeStruct((2, NP, _HALF), jnp.float32),
    )
    mid = pl.pallas_call(
        _mid_body,
        out_shape=jax.ShapeDtypeStruct((2, NP, _HALF), jnp.float32),
    )
    scatter = _make_scatter(NP, EP)

    src2 = src_p.reshape(-1, _K)
    dst2 = dst_p.reshape(-1, _K)

    g3 = enc(x_p, W_enc, b_enc.reshape(1, -1), W0, deg3)
    acc3 = scatter(g3, src2, dst2)
    for b_prev, W_next in ((b0, W1), (b1, W2), (b2, W3)):
        g3 = mid(acc3, deg3, b_prev.reshape(1, -1), W_next)
        acc3 = scatter(g3, src2, dst2)
    return _make_fin(N)(acc3, deg3, b3.reshape(1, -1))


# speedup vs baseline: 1.3385x; 1.1849x over previous
import jax
import jax.numpy as jnp
from jax import lax
from jax.experimental import pallas as pl
from jax.experimental.pallas import tpu as pltpu
from jax.experimental.pallas import tpu_sc as plsc

_NT = 16
_K = 128
_IC = 16
_HALF = 128


def _make_scatter(NP, EP):
    chunk = EP // _NT
    n_it = chunk // _K
    nq = n_it // _IC
    n2 = _IC // 2
    rpt = NP // _NT

    mesh = plsc.VectorSubcoreMesh(core_axis_name="c", subcore_axis_name="s")

    def body(g3, src_h, dst_h, acc3, isrc2, idst2, rows_a, rows_b,
             acc_sh, sem_i, sem_a, sem_b):
        c = lax.axis_index("c")
        s = lax.axis_index("s")
        r0 = s * rpt
        row0 = s * n_it

        pltpu.async_copy(src_h.at[pl.ds(row0, _IC)], isrc2.at[0], sem_i)
        pltpu.async_copy(dst_h.at[pl.ds(row0, _IC)], idst2.at[0], sem_i)
        pltpu.sync_copy(g3.at[c, pl.ds(r0, rpt)], acc_sh.at[pl.ds(r0, rpt)])
        plsc.subcore_barrier()
        gsrc = g3.at[c]

        def qloop(q, carry):
            p = lax.rem(q, 2)
            isrc = isrc2.at[p]
            idst = idst2.at[p]
            pltpu.make_async_copy(src_h.at[pl.ds(row0, _IC)],
                                  isrc, sem_i).wait()
            pltpu.make_async_copy(dst_h.at[pl.ds(row0, _IC)],
                                  idst, sem_i).wait()

            @pl.when(q + 1 < nq)
            def _():
                nxt = row0 + (q + 1) * _IC
                pltpu.async_copy(src_h.at[pl.ds(nxt, _IC)],
                                 isrc2.at[1 - p], sem_i)
                pltpu.async_copy(dst_h.at[pl.ds(nxt, _IC)],
                                 idst2.at[1 - p], sem_i)

            pltpu.async_copy(gsrc.at[isrc.at[0]], rows_a, sem_a)
            pltpu.async_copy(gsrc.at[isrc.at[1]], rows_b, sem_b)

            def it(j, carry2):
                i0 = 2 * j
                pltpu.make_async_copy(gsrc.at[isrc.at[i0]],
                                      rows_a, sem_a).wait()
                pltpu.sync_copy(rows_a, acc_sh.at[idst.at[i0]], add=True)

                @pl.when(j + 1 < n2)
                def _():
                    pltpu.async_copy(gsrc.at[isrc.at[i0 + 2]], rows_a, sem_a)

                pltpu.make_async_copy(gsrc.at[isrc.at[i0 + 1]],
                                      rows_b, sem_b).wait()
                pltpu.sync_copy(rows_b, acc_sh.at[idst.at[i0 + 1]], add=True)

                @pl.when(j + 1 < n2)
                def _():
                    pltpu.async_copy(gsrc.at[isrc.at[i0 + 3]], rows_b, sem_b)

                return carry2

            lax.fori_loop(0, n2, it, 0)
            return carry

        lax.fori_loop(0, nq, qloop, 0)
        plsc.subcore_barrier()
        pltpu.sync_copy(acc_sh.at[pl.ds(r0, rpt)], acc3.at[c, pl.ds(r0, rpt)])

    return pl.kernel(
        body,
        out_type=jax.ShapeDtypeStruct((2, NP, _HALF), jnp.float32),
        mesh=mesh,
        scratch_types=[
            pltpu.VMEM((2, _IC, _K), jnp.int32),
            pltpu.VMEM((2, _IC, _K), jnp.int32),
            pltpu.VMEM((_K, _HALF), jnp.float32),
            pltpu.VMEM((_K, _HALF), jnp.float32),
            pltpu.VMEM_SHARED((NP, _HALF), jnp.float32),
            pltpu.SemaphoreType.DMA,
            pltpu.SemaphoreType.DMA,
            pltpu.SemaphoreType.DMA,
        ],
    )


def _make_deg(NP, EP):
    rows_half = EP // _K // 2
    n_it = rows_half // _NT
    nq = n_it // _IC
    rpt = NP // _NT

    mesh = plsc.VectorSubcoreMesh(core_axis_name="c", subcore_axis_name="s")

    def body(dst_h, ones_h, zeros_h, deg3, ones_v, idst2, deg_sh, sem_i):
        c = lax.axis_index("c")
        s = lax.axis_index("s")
        r0 = s * rpt
        row0 = c * rows_half + s * n_it
        pltpu.async_copy(dst_h.at[pl.ds(row0, _IC)], idst2.at[0], sem_i)
        pltpu.sync_copy(zeros_h.at[pl.ds(r0, rpt)], deg_sh.at[pl.ds(r0, rpt)])
        pltpu.sync_copy(ones_h, ones_v)
        plsc.subcore_barrier()

        def qloop(q, carry):
            p = lax.rem(q, 2)
            idst = idst2.at[p]
            pltpu.make_async_copy(dst_h.at[pl.ds(row0, _IC)],
                                  idst, sem_i).wait()

            @pl.when(q + 1 < nq)
            def _():
                nxt = row0 + (q + 1) * _IC
                pltpu.async_copy(dst_h.at[pl.ds(nxt, _IC)],
                                 idst2.at[1 - p], sem_i)

            def it(i, carry2):
                pltpu.sync_copy(ones_v, deg_sh.at[idst.at[i]], add=True)
                return carry2

            lax.fori_loop(0, _IC, it, 0)
            return carry

        lax.fori_loop(0, nq, qloop, 0)
        plsc.subcore_barrier()
        pltpu.sync_copy(deg_sh.at[pl.ds(r0, rpt)], deg3.at[c, pl.ds(r0, rpt)])

    return pl.kernel(
        body,
        out_type=jax.ShapeDtypeStruct((2, NP, _HALF), jnp.float32),
        mesh=mesh,
        scratch_types=[
            pltpu.VMEM((_K, _HALF), jnp.float32),
            pltpu.VMEM((2, _IC, _K), jnp.int32),
            pltpu.VMEM_SHARED((NP, _HALF), jnp.float32),
            pltpu.SemaphoreType.DMA,
        ],
    )


def _dis(deg3_ref):
    return lax.rsqrt(deg3_ref[0, :, 0:1] + deg3_ref[1, :, 0:1] + 1.0)


def _enc_body(x_ref, we_ref, be_ref, w0_ref, deg3_ref, g3_ref):
    dis = _dis(deg3_ref)
    h = jnp.dot(x_ref[...], we_ref[...],
                preferred_element_type=jnp.float32) + be_ref[...]
    g = dis * jnp.dot(h, w0_ref[...], preferred_element_type=jnp.float32)
    g3_ref[0] = g[:, :_HALF]
    g3_ref[1] = g[:, _HALF:]


def _mid_body(acc3_ref, deg3_ref, b_ref, w_ref, g3_ref):
    dis = _dis(deg3_ref)
    h_a = jnp.maximum(dis * acc3_ref[0] + b_ref[:, :_HALF], 0.0)
    h_b = jnp.maximum(dis * acc3_ref[1] + b_ref[:, _HALF:], 0.0)
    h = jnp.concatenate([h_a, h_b], axis=1)
    g = dis * jnp.dot(h, w_ref[...], preferred_element_type=jnp.float32)
    g3_ref[0] = g[:, :_HALF]
    g3_ref[1] = g[:, _HALF:]


def _make_fin(N):
    def body(acc3_ref, deg3_ref, b_ref, out_ref):
        dis = _dis(deg3_ref)
        h_a = jnp.maximum(dis * acc3_ref[0] + b_ref[:, :_HALF], 0.0)
        h_b = jnp.maximum(dis * acc3_ref[1] + b_ref[:, _HALF:], 0.0)
        out_ref[...] = jnp.concatenate([h_a, h_b], axis=1)[:N]

    return pl.pallas_call(
        body,
        out_shape=jax.ShapeDtypeStruct((N, 2 * _HALF), jnp.float32),
    )


@jax.jit
def kernel(x, edge_index, W_enc, b_enc, W0, b0, W1, b1, W2, b2, W3, b3):
    N, D_IN = x.shape
    E = edge_index.shape[1]

    NP = ((N + 1 + 127) // 128) * 128
    grain = 2 * _NT * _K * _IC
    EP = ((E + grain - 1) // grain) * grain

    x_p = jnp.pad(x, ((0, NP - N), (0, 0)))
    pad = jnp.full((EP - E,), N, jnp.int32)
    src_p = jnp.concatenate([edge_index[0], pad])
    dst_p = jnp.concatenate([edge_index[1], pad])

    src2 = src_p.reshape(-1, _K)
    dst2 = dst_p.reshape(-1, _K)

    ones_h = jnp.ones((_K, _HALF), jnp.float32)
    zeros_h = jnp.zeros((NP, _HALF), jnp.float32)

    deg3 = _make_deg(NP, EP)(dst2, ones_h, zeros_h)

    enc = pl.pallas_call(
        _enc_body,
        out_shape=jax.ShapeDtypeStruct((2, NP, _HALF), jnp.float32),
    )
    mid = pl.pallas_call(
        _mid_body,
        out_shape=jax.ShapeDtypeStruct((2, NP, _HALF), jnp.float32),
    )
    scatter = _make_scatter(NP, EP)

    g3 = enc(x_p, W_enc, b_enc.reshape(1, -1), W0, deg3)
    acc3 = scatter(g3, src2, dst2)
    for b_prev, W_next in ((b0, W1), (b1, W2), (b2, W3)):
        g3 = mid(acc3, deg3, b_prev.reshape(1, -1), W_next)
        acc3 = scatter(g3, src2, dst2)
    return _make_fin(N)(acc3, deg3, b3.reshape(1, -1))
